# baseline (speedup 1.0000x reference)
import functools

import jax
import jax.numpy as jnp
from jax import lax
from jax.experimental import pallas as pl
from jax.experimental.pallas import tpu as pltpu
from jax.experimental.pallas import tpu_sc as plsc

N0 = 10000
N1 = 5000
N2 = 2000
E0 = 320000
E1 = 160000
D = 128
H = 256
O = 64

NC = 2
NS = 16
NW = NC * NS
L = 16

N1P = 5120
N2P = 2048
DEGW = 8
ZR = 32

CHUNK = 112
GS = 5
SPLIT0 = (65, 115)
SPLIT1 = (35, 55)
DG = 9
ITERS0 = sum(SPLIT0) // 2
ITERS1 = sum(SPLIT1) // 2


def _mesh():
    return plsc.VectorSubcoreMesh(core_axis_name="c", subcore_axis_name="s")


def _fill_rows(ref, n_rows, width, value):
    v16 = jnp.full((L,), value, jnp.float32)
    if width >= L:
        per_row = width // L

        def body(i, _):
            ref[i // per_row, pl.ds((i % per_row) * L, L)] = v16
            return 0
        lax.fori_loop(0, n_rows * per_row, body, 0)
    else:
        rows_per_store = L // width

        def body(i, _):
            ref[i // rows_per_store,
                pl.ds((i % rows_per_store) * width, width)] = v16[:width]
            return 0
        lax.fori_loop(0, n_rows * rows_per_store, body, 0)


@functools.lru_cache(maxsize=None)
def _make_segsum(n_tgt_pad: int, split: tuple):
    ca, cb = split
    assert ca % GS == 0 and cb % GS == 0
    rows_per_sub = n_tgt_pad // NS
    assert rows_per_sub % ZR == 0

    @functools.partial(
        pl.kernel,
        mesh=_mesh(),
        out_type=jax.ShapeDtypeStruct((NC, n_tgt_pad, D), jnp.float32),
        scratch_types=[
            [pltpu.VMEM((CHUNK,), jnp.int32)] * GS,
            [pltpu.VMEM((CHUNK,), jnp.int32)] * GS,
            [pltpu.VMEM((CHUNK, D), jnp.float32)] * GS,
            pltpu.VMEM((ZR, D), jnp.float32),
            pltpu.VMEM_SHARED((n_tgt_pad, D), jnp.float32),
            pltpu.SemaphoreType.DMA,
            pltpu.SemaphoreType.DMA,
            pltpu.SemaphoreType.DMA,
        ],
    )
    def k(table, eidx, acc_out,
          src_v, dst_v, rows_v, zrow_v, acc_sh, sem_i, sem_g, sem_s):
        cid = lax.axis_index("c")
        sid = lax.axis_index("s")

        _fill_rows(zrow_v, ZR, D, 0.0)

        base_r = sid * rows_per_sub

        def zero_acc(i, _):
            pltpu.async_copy(zrow_v, acc_sh.at[pl.ds(base_r + i * ZR, ZR)],
                             sem_s)
            return 0
        lax.fori_loop(0, rows_per_sub // ZR, zero_acc, 0)

        def zero_drain(i, _):
            pltpu.make_async_copy(
                zrow_v, acc_sh.at[pl.ds(base_r, ZR)], sem_s).wait()
            return 0
        lax.fori_loop(0, rows_per_sub // ZR, zero_drain, 0)

        plsc.subcore_barrier()

        n_groups = jnp.where(cid == 0, ca // GS, cb // GS)
        wbase = (cid * (NS * ca) + sid * jnp.where(cid == 0, ca, cb)) \
            * 2 * CHUNK

        def body(g, _):
            t0 = g * GS
            ics = []
            for j in range(GS):
                cb = wbase + (t0 + j) * 2 * CHUNK
                ics.append(pltpu.async_copy(
                    eidx.at[pl.ds(cb, CHUNK)], src_v[j], sem_i))
                ics.append(pltpu.async_copy(
                    eidx.at[pl.ds(cb + CHUNK, CHUNK)], dst_v[j], sem_i))
            for cp in ics:
                cp.wait()
            cps = [pltpu.async_copy(table.at[src_v[j]], rows_v[j], sem_g)
                   for j in range(GS)]
            for cp in cps:
                cp.wait()
            scs = [pltpu.async_copy(rows_v[j], acc_sh.at[dst_v[j]], sem_s,
                                    add=True)
                   for j in range(GS)]
            for cp in scs:
                cp.wait()
            return 0
        lax.fori_loop(0, n_groups, body, 0)

        plsc.subcore_barrier()

        pltpu.sync_copy(acc_sh.at[pl.ds(base_r, rows_per_sub)],
                        acc_out.at[cid, pl.ds(base_r, rows_per_sub)])

    return k


@functools.lru_cache(maxsize=None)
def _make_degrees():
    r0 = N1P // NS
    r1 = N2P // NS

    @functools.partial(
        pl.kernel,
        mesh=_mesh(),
        out_type=[
            jax.ShapeDtypeStruct((NC, N1P, DEGW), jnp.float32),
            jax.ShapeDtypeStruct((NC, N2P, DEGW), jnp.float32),
        ],
        scratch_types=[
            [pltpu.VMEM((CHUNK,), jnp.int32)] * DG,
            pltpu.VMEM((CHUNK, DEGW), jnp.float32),
            pltpu.VMEM((ZR, DEGW), jnp.float32),
            pltpu.VMEM_SHARED((N1P, DEGW), jnp.float32),
            pltpu.VMEM_SHARED((N2P, DEGW), jnp.float32),
            pltpu.SemaphoreType.DMA,
            pltpu.SemaphoreType.DMA,
        ],
    )
    def k(edst0, edst1, deg0_out, deg1_out,
          dst_v, ones_v, zdeg_v, deg0_sh, deg1_sh, sem_i, sem_s):
        cid = lax.axis_index("c")
        sid = lax.axis_index("s")
        wid = sid * NC + cid

        _fill_rows(ones_v, CHUNK, DEGW, 1.0)
        _fill_rows(zdeg_v, ZR, DEGW, 0.0)

        def zero0(i, _):
            pltpu.async_copy(zdeg_v, deg0_sh.at[pl.ds(sid * r0 + i * ZR, ZR)],
                             sem_s)
            return 0
        lax.fori_loop(0, r0 // ZR, zero0, 0)

        def zero1(i, _):
            pltpu.async_copy(zdeg_v, deg1_sh.at[pl.ds(sid * r1 + i * ZR, ZR)],
                             sem_s)
            return 0
        lax.fori_loop(0, r1 // ZR, zero1, 0)

        def zero_drain(i, _):
            pltpu.make_async_copy(
                zdeg_v, deg0_sh.at[pl.ds(sid * r0, ZR)], sem_s).wait()
            return 0
        lax.fori_loop(0, r0 // ZR + r1 // ZR, zero_drain, 0)

        plsc.subcore_barrier()

        def layer(edst, deg_sh, iters):
            wbase = wid * iters * CHUNK
            n_groups = iters // DG

            def body(g, _):
                t0 = g * DG
                ics = [pltpu.async_copy(
                    edst.at[pl.ds(wbase + (t0 + j) * CHUNK, CHUNK)],
                    dst_v[j], sem_i)
                    for j in range(DG)]
                for cp in ics:
                    cp.wait()
                scs = [pltpu.async_copy(ones_v, deg_sh.at[dst_v[j]], sem_s,
                                        add=True)
                       for j in range(DG)]
                for cp in scs:
                    cp.wait()
                return 0
            lax.fori_loop(0, n_groups, body, 0)

        layer(edst0, deg0_sh, ITERS0)
        layer(edst1, deg1_sh, ITERS1)

        plsc.subcore_barrier()

        pltpu.sync_copy(deg0_sh.at[pl.ds(sid * r0, r0)],
                        deg0_out.at[cid, pl.ds(sid * r0, r0)])
        pltpu.sync_copy(deg1_sh.at[pl.ds(sid * r1, r1)],
                        deg1_out.at[cid, pl.ds(sid * r1, r1)])

    return k


def _prep_edges(edge_index, split, pad_dst):
    e = edge_index.shape[1]
    iters = sum(split) // 2
    e_pad = NW * iters * CHUNK
    src = edge_index[0].astype(jnp.int32)
    dst = edge_index[1].astype(jnp.int32)
    src = jnp.concatenate([src, jnp.zeros((e_pad - e,), jnp.int32)])
    dst = jnp.concatenate([dst, jnp.full((e_pad - e,), pad_dst, jnp.int32)])
    both = (jnp.stack([src, dst], 0)
            .reshape(2, NW * iters, CHUNK)
            .transpose(1, 0, 2)
            .reshape(-1))
    dst_only = (dst.reshape(NW, iters, CHUNK)
                .reshape(-1))
    return both, dst_only


def _mlp1_body(a0, a1, d0, d1, w, b, out):
    deg = d0[:, 0:1] + d1[:, 0:1]
    a = (a0[...] + a1[...]) / jnp.maximum(deg, 1.0)
    y = jnp.dot(a, w[...], preferred_element_type=jnp.float32) + b[...]
    n = jnp.sqrt(jnp.sum(y * y, axis=-1, keepdims=True))
    out[...] = y / jnp.maximum(n, 1e-12)


def _mlp1(acc, deg, W1, b1):
    BR = 640
    grid = N1P // BR
    return pl.pallas_call(
        _mlp1_body,
        grid=(grid,),
        in_specs=[
            pl.BlockSpec((BR, D), lambda i: (i, 0)),
            pl.BlockSpec((BR, D), lambda i: (i, 0)),
            pl.BlockSpec((BR, DEGW), lambda i: (i, 0)),
            pl.BlockSpec((BR, DEGW), lambda i: (i, 0)),
            pl.BlockSpec((D, D), lambda i: (0, 0)),
            pl.BlockSpec((1, D), lambda i: (0, 0)),
        ],
        out_specs=pl.BlockSpec((BR, D), lambda i: (i, 0)),
        out_shape=jax.ShapeDtypeStruct((N1P, D), jnp.float32),
    )(acc[0], acc[1], deg[0], deg[1], W1, b1)


def _mlp2_body(a0, a1, d0, d1, wa, ba, wb, bb, out):
    deg = d0[:, 0:1] + d1[:, 0:1]
    a = (a0[...] + a1[...]) / jnp.maximum(deg, 1.0)
    y = jnp.dot(a, wa[...], preferred_element_type=jnp.float32) + ba[...]
    y = jnp.maximum(y, 0.0)
    z = jnp.dot(y, wb[...], preferred_element_type=jnp.float32) + bb[...]
    n = jnp.sqrt(jnp.sum(z * z, axis=-1, keepdims=True))
    out[...] = z / jnp.maximum(n, 1e-12)


def _mlp2(acc, deg, W2a, b2a, W2b, b2b):
    BR = 512
    grid = N2P // BR
    return pl.pallas_call(
        _mlp2_body,
        grid=(grid,),
        in_specs=[
            pl.BlockSpec((BR, D), lambda i: (i, 0)),
            pl.BlockSpec((BR, D), lambda i: (i, 0)),
            pl.BlockSpec((BR, DEGW), lambda i: (i, 0)),
            pl.BlockSpec((BR, DEGW), lambda i: (i, 0)),
            pl.BlockSpec((D, H), lambda i: (0, 0)),
            pl.BlockSpec((1, H), lambda i: (0, 0)),
            pl.BlockSpec((H, O), lambda i: (0, 0)),
            pl.BlockSpec((1, O), lambda i: (0, 0)),
        ],
        out_specs=pl.BlockSpec((BR, O), lambda i: (i, 0)),
        out_shape=jax.ShapeDtypeStruct((N2P, O), jnp.float32),
    )(acc[0], acc[1], deg[0], deg[1], W2a, b2a, W2b, b2b)


def kernel(x, edge_index0, edge_index1, W1, b1, W2a, b2a, W2b, b2b):
    eidx0, edst0 = _prep_edges(edge_index0, SPLIT0, N1P - 1)
    eidx1, edst1 = _prep_edges(edge_index1, SPLIT1, N2P - 1)

    deg0, deg1 = _make_degrees()(edst0, edst1)
    acc0 = _make_segsum(N1P, SPLIT0)(x, eidx0)
    h = _mlp1(acc0, deg0, W1, b1.reshape(1, D))
    acc1 = _make_segsum(N2P, SPLIT1)(h, eidx1)
    out = _mlp2(acc1, deg1, W2a, b2a.reshape(1, H), W2b, b2b.reshape(1, O))
    return out[:N2]


# speedup vs baseline: 1.2035x; 1.2035x over previous
import functools

import jax
import jax.numpy as jnp
from jax import lax
from jax.experimental import pallas as pl
from jax.experimental.pallas import tpu as pltpu
from jax.experimental.pallas import tpu_sc as plsc

N0 = 10000
N1 = 5000
N2 = 2000
E0 = 320000
E1 = 160000
D = 128
H = 256
O = 64

NC = 2
NS = 16
NW = NC * NS
L = 16

N1P = 5120
N2P = 2048
DEGW = 8
ZR = 32

CHUNK = 112
GS = 5
SPLIT0 = (115, 65)
SPLIT1 = (55, 35)
DG = 9
ITERS0 = sum(SPLIT0) // 2
ITERS1 = sum(SPLIT1) // 2


def _mesh():
    return plsc.VectorSubcoreMesh(core_axis_name="c", subcore_axis_name="s")


def _fill_rows(ref, n_rows, width, value):
    v16 = jnp.full((L,), value, jnp.float32)
    if width >= L:
        per_row = width // L

        def body(i, _):
            ref[i // per_row, pl.ds((i % per_row) * L, L)] = v16
            return 0
        lax.fori_loop(0, n_rows * per_row, body, 0)
    else:
        rows_per_store = L // width

        def body(i, _):
            ref[i // rows_per_store,
                pl.ds((i % rows_per_store) * width, width)] = v16[:width]
            return 0
        lax.fori_loop(0, n_rows * rows_per_store, body, 0)


@functools.lru_cache(maxsize=None)
def _make_segsum(n_tgt_pad: int, split: tuple):
    ca, cb = split
    assert ca % GS == 0 and cb % GS == 0
    rows_per_sub = n_tgt_pad // NS
    assert rows_per_sub % ZR == 0

    @functools.partial(
        pl.kernel,
        mesh=_mesh(),
        out_type=jax.ShapeDtypeStruct((NC, n_tgt_pad, D), jnp.float32),
        scratch_types=[
            [pltpu.VMEM((CHUNK,), jnp.int32)] * GS,
            [pltpu.VMEM((CHUNK,), jnp.int32)] * GS,
            [pltpu.VMEM((CHUNK, D), jnp.float32)] * GS,
            pltpu.VMEM((ZR, D), jnp.float32),
            pltpu.VMEM_SHARED((n_tgt_pad, D), jnp.float32),
            pltpu.SemaphoreType.DMA,
            pltpu.SemaphoreType.DMA,
            pltpu.SemaphoreType.DMA,
        ],
    )
    def k(table, eidx, acc_out,
          src_v, dst_v, rows_v, zrow_v, acc_sh, sem_i, sem_g, sem_s):
        cid = lax.axis_index("c")
        sid = lax.axis_index("s")

        _fill_rows(zrow_v, ZR, D, 0.0)

        base_r = sid * rows_per_sub

        def zero_acc(i, _):
            pltpu.async_copy(zrow_v, acc_sh.at[pl.ds(base_r + i * ZR, ZR)],
                             sem_s)
            return 0
        lax.fori_loop(0, rows_per_sub // ZR, zero_acc, 0)

        def zero_drain(i, _):
            pltpu.make_async_copy(
                zrow_v, acc_sh.at[pl.ds(base_r, ZR)], sem_s).wait()
            return 0
        lax.fori_loop(0, rows_per_sub // ZR, zero_drain, 0)

        plsc.subcore_barrier()

        n_groups = jnp.where(cid == 0, ca // GS, cb // GS)
        wbase = (cid * (NS * ca) + sid * jnp.where(cid == 0, ca, cb)) \
            * 2 * CHUNK

        def body(g, _):
            t0 = g * GS
            ics = []
            for j in range(GS):
                cb = wbase + (t0 + j) * 2 * CHUNK
                ics.append(pltpu.async_copy(
                    eidx.at[pl.ds(cb, CHUNK)], src_v[j], sem_i))
                ics.append(pltpu.async_copy(
                    eidx.at[pl.ds(cb + CHUNK, CHUNK)], dst_v[j], sem_i))
            for cp in ics:
                cp.wait()
            cps = [pltpu.async_copy(table.at[src_v[j]], rows_v[j], sem_g)
                   for j in range(GS)]
            for cp in cps:
                cp.wait()
            scs = [pltpu.async_copy(rows_v[j], acc_sh.at[dst_v[j]], sem_s,
                                    add=True)
                   for j in range(GS)]
            for cp in scs:
                cp.wait()
            return 0
        lax.fori_loop(0, n_groups, body, 0)

        plsc.subcore_barrier()

        pltpu.sync_copy(acc_sh.at[pl.ds(base_r, rows_per_sub)],
                        acc_out.at[cid, pl.ds(base_r, rows_per_sub)])

    return k


@functools.lru_cache(maxsize=None)
def _make_degrees():
    r0 = N1P // NS
    r1 = N2P // NS

    @functools.partial(
        pl.kernel,
        mesh=_mesh(),
        out_type=[
            jax.ShapeDtypeStruct((NC, N1P, DEGW), jnp.float32),
            jax.ShapeDtypeStruct((NC, N2P, DEGW), jnp.float32),
        ],
        scratch_types=[
            [pltpu.VMEM((CHUNK,), jnp.int32)] * DG,
            pltpu.VMEM((CHUNK, DEGW), jnp.float32),
            pltpu.VMEM((ZR, DEGW), jnp.float32),
            pltpu.VMEM_SHARED((N1P, DEGW), jnp.float32),
            pltpu.VMEM_SHARED((N2P, DEGW), jnp.float32),
            pltpu.SemaphoreType.DMA,
            pltpu.SemaphoreType.DMA,
        ],
    )
    def k(edst0, edst1, deg0_out, deg1_out,
          dst_v, ones_v, zdeg_v, deg0_sh, deg1_sh, sem_i, sem_s):
        cid = lax.axis_index("c")
        sid = lax.axis_index("s")
        wid = sid * NC + cid

        _fill_rows(ones_v, CHUNK, DEGW, 1.0)
        _fill_rows(zdeg_v, ZR, DEGW, 0.0)

        def zero0(i, _):
            pltpu.async_copy(zdeg_v, deg0_sh.at[pl.ds(sid * r0 + i * ZR, ZR)],
                             sem_s)
            return 0
        lax.fori_loop(0, r0 // ZR, zero0, 0)

        def zero1(i, _):
            pltpu.async_copy(zdeg_v, deg1_sh.at[pl.ds(sid * r1 + i * ZR, ZR)],
                             sem_s)
            return 0
        lax.fori_loop(0, r1 // ZR, zero1, 0)

        def zero_drain(i, _):
            pltpu.make_async_copy(
                zdeg_v, deg0_sh.at[pl.ds(sid * r0, ZR)], sem_s).wait()
            return 0
        lax.fori_loop(0, r0 // ZR + r1 // ZR, zero_drain, 0)

        plsc.subcore_barrier()

        def layer(edst, deg_sh, iters):
            wbase = wid * iters * CHUNK
            n_groups = iters // DG

            def body(g, _):
                t0 = g * DG
                ics = [pltpu.async_copy(
                    edst.at[pl.ds(wbase + (t0 + j) * CHUNK, CHUNK)],
                    dst_v[j], sem_i)
                    for j in range(DG)]
                for cp in ics:
                    cp.wait()
                scs = [pltpu.async_copy(ones_v, deg_sh.at[dst_v[j]], sem_s,
                                        add=True)
                       for j in range(DG)]
                for cp in scs:
                    cp.wait()
                return 0
            lax.fori_loop(0, n_groups, body, 0)

        layer(edst0, deg0_sh, ITERS0)
        layer(edst1, deg1_sh, ITERS1)

        plsc.subcore_barrier()

        pltpu.sync_copy(deg0_sh.at[pl.ds(sid * r0, r0)],
                        deg0_out.at[cid, pl.ds(sid * r0, r0)])
        pltpu.sync_copy(deg1_sh.at[pl.ds(sid * r1, r1)],
                        deg1_out.at[cid, pl.ds(sid * r1, r1)])

    return k


def _prep_edges(edge_index, split, pad_dst):
    e = edge_index.shape[1]
    iters = sum(split) // 2
    e_pad = NW * iters * CHUNK
    src = edge_index[0].astype(jnp.int32)
    dst = edge_index[1].astype(jnp.int32)
    src = jnp.concatenate([src, jnp.zeros((e_pad - e,), jnp.int32)])
    dst = jnp.concatenate([dst, jnp.full((e_pad - e,), pad_dst, jnp.int32)])
    both = (jnp.stack([src, dst], 0)
            .reshape(2, NW * iters, CHUNK)
            .transpose(1, 0, 2)
            .reshape(-1))
    dst_only = (dst.reshape(NW, iters, CHUNK)
                .reshape(-1))
    return both, dst_only


def _mlp1_body(a0, a1, d0, d1, w, b, out):
    deg = d0[:, 0:1] + d1[:, 0:1]
    a = (a0[...] + a1[...]) / jnp.maximum(deg, 1.0)
    y = jnp.dot(a, w[...], preferred_element_type=jnp.float32) + b[...]
    n = jnp.sqrt(jnp.sum(y * y, axis=-1, keepdims=True))
    out[...] = y / jnp.maximum(n, 1e-12)


def _mlp1(acc, deg, W1, b1):
    BR = 640
    grid = N1P // BR
    return pl.pallas_call(
        _mlp1_body,
        grid=(grid,),
        in_specs=[
            pl.BlockSpec((BR, D), lambda i: (i, 0)),
            pl.BlockSpec((BR, D), lambda i: (i, 0)),
            pl.BlockSpec((BR, DEGW), lambda i: (i, 0)),
            pl.BlockSpec((BR, DEGW), lambda i: (i, 0)),
            pl.BlockSpec((D, D), lambda i: (0, 0)),
            pl.BlockSpec((1, D), lambda i: (0, 0)),
        ],
        out_specs=pl.BlockSpec((BR, D), lambda i: (i, 0)),
        out_shape=jax.ShapeDtypeStruct((N1P, D), jnp.float32),
    )(acc[0], acc[1], deg[0], deg[1], W1, b1)


def _mlp2_body(a0, a1, d0, d1, wa, ba, wb, bb, out):
    deg = d0[:, 0:1] + d1[:, 0:1]
    a = (a0[...] + a1[...]) / jnp.maximum(deg, 1.0)
    y = jnp.dot(a, wa[...], preferred_element_type=jnp.float32) + ba[...]
    y = jnp.maximum(y, 0.0)
    z = jnp.dot(y, wb[...], preferred_element_type=jnp.float32) + bb[...]
    n = jnp.sqrt(jnp.sum(z * z, axis=-1, keepdims=True))
    out[...] = z / jnp.maximum(n, 1e-12)


def _mlp2(acc, deg, W2a, b2a, W2b, b2b):
    BR = 512
    grid = N2P // BR
    return pl.pallas_call(
        _mlp2_body,
        grid=(grid,),
        in_specs=[
            pl.BlockSpec((BR, D), lambda i: (i, 0)),
            pl.BlockSpec((BR, D), lambda i: (i, 0)),
            pl.BlockSpec((BR, DEGW), lambda i: (i, 0)),
            pl.BlockSpec((BR, DEGW), lambda i: (i, 0)),
            pl.BlockSpec((D, H), lambda i: (0, 0)),
            pl.BlockSpec((1, H), lambda i: (0, 0)),
            pl.BlockSpec((H, O), lambda i: (0, 0)),
            pl.BlockSpec((1, O), lambda i: (0, 0)),
        ],
        out_specs=pl.BlockSpec((BR, O), lambda i: (i, 0)),
        out_shape=jax.ShapeDtypeStruct((N2P, O), jnp.float32),
    )(acc[0], acc[1], deg[0], deg[1], W2a, b2a, W2b, b2b)


def kernel(x, edge_index0, edge_index1, W1, b1, W2a, b2a, W2b, b2b):
    eidx0, edst0 = _prep_edges(edge_index0, SPLIT0, N1P - 1)
    eidx1, edst1 = _prep_edges(edge_index1, SPLIT1, N2P - 1)

    deg0, deg1 = _make_degrees()(edst0, edst1)
    acc0 = _make_segsum(N1P, SPLIT0)(x, eidx0)
    h = _mlp1(acc0, deg0, W1, b1.reshape(1, D))
    acc1 = _make_segsum(N2P, SPLIT1)(h, eidx1)
    out = _mlp2(acc1, deg1, W2a, b2a.reshape(1, H), W2b, b2b.reshape(1, O))
    return out[:N2]


# speedup vs baseline: 1.2187x; 1.0127x over previous
import functools

import jax
import jax.numpy as jnp
from jax import lax
from jax.experimental import pallas as pl
from jax.experimental.pallas import tpu as pltpu
from jax.experimental.pallas import tpu_sc as plsc

N0 = 10000
N1 = 5000
N2 = 2000
E0 = 320000
E1 = 160000
D = 128
H = 256
O = 64

NC = 2
NS = 16
NW = NC * NS
L = 16

N1P = 5120
N2P = 2048
DEGW = 8
ZR = 8

CHUNK = 112
GS = 3
SPLIT0 = (108, 72)
SPLIT1 = (54, 36)
DG = 9
ITERS0 = sum(SPLIT0) // 2
ITERS1 = sum(SPLIT1) // 2


def _mesh():
    return plsc.VectorSubcoreMesh(core_axis_name="c", subcore_axis_name="s")


def _fill_rows(ref, n_rows, width, value):
    v16 = jnp.full((L,), value, jnp.float32)
    if width >= L:
        per_row = width // L

        def body(i, _):
            ref[i // per_row, pl.ds((i % per_row) * L, L)] = v16
            return 0
        lax.fori_loop(0, n_rows * per_row, body, 0)
    else:
        rows_per_store = L // width

        def body(i, _):
            ref[i // rows_per_store,
                pl.ds((i % rows_per_store) * width, width)] = v16[:width]
            return 0
        lax.fori_loop(0, n_rows * rows_per_store, body, 0)


@functools.lru_cache(maxsize=None)
def _make_segsum(n_tgt_pad: int, split: tuple):
    ca, cb = split
    assert ca % GS == 0 and cb % GS == 0
    rows_per_sub = n_tgt_pad // NS
    assert rows_per_sub % ZR == 0

    @functools.partial(
        pl.kernel,
        mesh=_mesh(),
        out_type=jax.ShapeDtypeStruct((NC, n_tgt_pad, D), jnp.float32),
        scratch_types=[
            [pltpu.VMEM((CHUNK,), jnp.int32)] * (2 * GS),
            [pltpu.VMEM((CHUNK,), jnp.int32)] * (2 * GS),
            [pltpu.VMEM((CHUNK, D), jnp.float32)] * (2 * GS),
            pltpu.VMEM((ZR, D), jnp.float32),
            pltpu.VMEM_SHARED((n_tgt_pad, D), jnp.float32),
            pltpu.SemaphoreType.DMA,
            pltpu.SemaphoreType.DMA,
            pltpu.SemaphoreType.DMA,
            pltpu.SemaphoreType.DMA,
        ],
    )
    def k(table, eidx, acc_out,
          src_v, dst_v, rows_v, zrow_v, acc_sh, sem_i, sem_ga, sem_gb,
          sem_s):
        cid = lax.axis_index("c")
        sid = lax.axis_index("s")

        _fill_rows(zrow_v, ZR, D, 0.0)

        base_r = sid * rows_per_sub

        def zero_acc(i, _):
            pltpu.async_copy(zrow_v, acc_sh.at[pl.ds(base_r + i * ZR, ZR)],
                             sem_s)
            return 0
        lax.fori_loop(0, rows_per_sub // ZR, zero_acc, 0)

        def zero_drain(i, _):
            pltpu.make_async_copy(
                zrow_v, acc_sh.at[pl.ds(base_r, ZR)], sem_s).wait()
            return 0
        lax.fori_loop(0, rows_per_sub // ZR, zero_drain, 0)

        plsc.subcore_barrier()

        n_pairs = jnp.where(cid == 0, ca // (2 * GS), cb // (2 * GS))
        wbase = (cid * (NS * ca) + sid * jnp.where(cid == 0, ca, cb)) \
            * 2 * CHUNK

        def stage(g, par):
            cps = []
            for j in range(GS):
                cb = wbase + (g * GS + j) * 2 * CHUNK
                cps.append(pltpu.async_copy(
                    eidx.at[pl.ds(cb, CHUNK)], src_v[par * GS + j], sem_i))
                cps.append(pltpu.async_copy(
                    eidx.at[pl.ds(cb + CHUNK, CHUNK)], dst_v[par * GS + j],
                    sem_i))
            return cps

        def body(u, _):
            g = 2 * u
            for cp in stage(g, 0):
                cp.wait()
            ga = [pltpu.async_copy(table.at[src_v[j]], rows_v[j], sem_ga)
                  for j in range(GS)]
            sb = stage(g + 1, 1)
            for cp in sb:
                cp.wait()
            gb = [pltpu.async_copy(table.at[src_v[GS + j]], rows_v[GS + j],
                                   sem_gb)
                  for j in range(GS)]
            for cp in ga:
                cp.wait()
            sca = [pltpu.async_copy(rows_v[j], acc_sh.at[dst_v[j]], sem_s,
                                    add=True)
                   for j in range(GS)]
            for cp in gb:
                cp.wait()
            for cp in sca:
                cp.wait()
            scb = [pltpu.async_copy(rows_v[GS + j],
                                    acc_sh.at[dst_v[GS + j]], sem_s,
                                    add=True)
                   for j in range(GS)]
            for cp in scb:
                cp.wait()
            return 0
        lax.fori_loop(0, n_pairs, body, 0)

        plsc.subcore_barrier()

        pltpu.sync_copy(acc_sh.at[pl.ds(base_r, rows_per_sub)],
                        acc_out.at[cid, pl.ds(base_r, rows_per_sub)])

    return k


@functools.lru_cache(maxsize=None)
def _make_degrees():
    r0 = N1P // NS
    r1 = N2P // NS

    @functools.partial(
        pl.kernel,
        mesh=_mesh(),
        out_type=[
            jax.ShapeDtypeStruct((NC, N1P, DEGW), jnp.float32),
            jax.ShapeDtypeStruct((NC, N2P, DEGW), jnp.float32),
        ],
        scratch_types=[
            [pltpu.VMEM((CHUNK,), jnp.int32)] * DG,
            pltpu.VMEM((CHUNK, DEGW), jnp.float32),
            pltpu.VMEM((ZR, DEGW), jnp.float32),
            pltpu.VMEM_SHARED((N1P, DEGW), jnp.float32),
            pltpu.VMEM_SHARED((N2P, DEGW), jnp.float32),
            pltpu.SemaphoreType.DMA,
            pltpu.SemaphoreType.DMA,
        ],
    )
    def k(edst0, edst1, deg0_out, deg1_out,
          dst_v, ones_v, zdeg_v, deg0_sh, deg1_sh, sem_i, sem_s):
        cid = lax.axis_index("c")
        sid = lax.axis_index("s")
        wid = sid * NC + cid

        _fill_rows(ones_v, CHUNK, DEGW, 1.0)
        _fill_rows(zdeg_v, ZR, DEGW, 0.0)

        def zero0(i, _):
            pltpu.async_copy(zdeg_v, deg0_sh.at[pl.ds(sid * r0 + i * ZR, ZR)],
                             sem_s)
            return 0
        lax.fori_loop(0, r0 // ZR, zero0, 0)

        def zero1(i, _):
            pltpu.async_copy(zdeg_v, deg1_sh.at[pl.ds(sid * r1 + i * ZR, ZR)],
                             sem_s)
            return 0
        lax.fori_loop(0, r1 // ZR, zero1, 0)

        def zero_drain(i, _):
            pltpu.make_async_copy(
                zdeg_v, deg0_sh.at[pl.ds(sid * r0, ZR)], sem_s).wait()
            return 0
        lax.fori_loop(0, r0 // ZR + r1 // ZR, zero_drain, 0)

        plsc.subcore_barrier()

        def layer(edst, deg_sh, iters):
            wbase = wid * iters * CHUNK
            n_groups = iters // DG

            def body(g, _):
                t0 = g * DG
                ics = [pltpu.async_copy(
                    edst.at[pl.ds(wbase + (t0 + j) * CHUNK, CHUNK)],
                    dst_v[j], sem_i)
                    for j in range(DG)]
                for cp in ics:
                    cp.wait()
                scs = [pltpu.async_copy(ones_v, deg_sh.at[dst_v[j]], sem_s,
                                        add=True)
                       for j in range(DG)]
                for cp in scs:
                    cp.wait()
                return 0
            lax.fori_loop(0, n_groups, body, 0)

        layer(edst0, deg0_sh, ITERS0)
        layer(edst1, deg1_sh, ITERS1)

        plsc.subcore_barrier()

        pltpu.sync_copy(deg0_sh.at[pl.ds(sid * r0, r0)],
                        deg0_out.at[cid, pl.ds(sid * r0, r0)])
        pltpu.sync_copy(deg1_sh.at[pl.ds(sid * r1, r1)],
                        deg1_out.at[cid, pl.ds(sid * r1, r1)])

    return k


def _prep_edges(edge_index, split, pad_dst):
    e = edge_index.shape[1]
    iters = sum(split) // 2
    e_pad = NW * iters * CHUNK
    src = edge_index[0].astype(jnp.int32)
    dst = edge_index[1].astype(jnp.int32)
    src = jnp.concatenate([src, jnp.zeros((e_pad - e,), jnp.int32)])
    dst = jnp.concatenate([dst, jnp.full((e_pad - e,), pad_dst, jnp.int32)])
    both = (jnp.stack([src, dst], 0)
            .reshape(2, NW * iters, CHUNK)
            .transpose(1, 0, 2)
            .reshape(-1))
    dst_only = (dst.reshape(NW, iters, CHUNK)
                .reshape(-1))
    return both, dst_only


def _mlp1_body(a0, a1, d0, d1, w, b, out):
    deg = d0[:, 0:1] + d1[:, 0:1]
    a = (a0[...] + a1[...]) / jnp.maximum(deg, 1.0)
    y = jnp.dot(a, w[...], preferred_element_type=jnp.float32) + b[...]
    n = jnp.sqrt(jnp.sum(y * y, axis=-1, keepdims=True))
    out[...] = y / jnp.maximum(n, 1e-12)


def _mlp1(acc, deg, W1, b1):
    BR = 640
    grid = N1P // BR
    return pl.pallas_call(
        _mlp1_body,
        grid=(grid,),
        in_specs=[
            pl.BlockSpec((BR, D), lambda i: (i, 0)),
            pl.BlockSpec((BR, D), lambda i: (i, 0)),
            pl.BlockSpec((BR, DEGW), lambda i: (i, 0)),
            pl.BlockSpec((BR, DEGW), lambda i: (i, 0)),
            pl.BlockSpec((D, D), lambda i: (0, 0)),
            pl.BlockSpec((1, D), lambda i: (0, 0)),
        ],
        out_specs=pl.BlockSpec((BR, D), lambda i: (i, 0)),
        out_shape=jax.ShapeDtypeStruct((N1P, D), jnp.float32),
    )(acc[0], acc[1], deg[0], deg[1], W1, b1)


def _mlp2_body(a0, a1, d0, d1, wa, ba, wb, bb, out):
    deg = d0[:, 0:1] + d1[:, 0:1]
    a = (a0[...] + a1[...]) / jnp.maximum(deg, 1.0)
    y = jnp.dot(a, wa[...], preferred_element_type=jnp.float32) + ba[...]
    y = jnp.maximum(y, 0.0)
    z = jnp.dot(y, wb[...], preferred_element_type=jnp.float32) + bb[...]
    n = jnp.sqrt(jnp.sum(z * z, axis=-1, keepdims=True))
    out[...] = z / jnp.maximum(n, 1e-12)


def _mlp2(acc, deg, W2a, b2a, W2b, b2b):
    BR = 512
    grid = N2P // BR
    return pl.pallas_call(
        _mlp2_body,
        grid=(grid,),
        in_specs=[
            pl.BlockSpec((BR, D), lambda i: (i, 0)),
            pl.BlockSpec((BR, D), lambda i: (i, 0)),
            pl.BlockSpec((BR, DEGW), lambda i: (i, 0)),
            pl.BlockSpec((BR, DEGW), lambda i: (i, 0)),
            pl.BlockSpec((D, H), lambda i: (0, 0)),
            pl.BlockSpec((1, H), lambda i: (0, 0)),
            pl.BlockSpec((H, O), lambda i: (0, 0)),
            pl.BlockSpec((1, O), lambda i: (0, 0)),
        ],
        out_specs=pl.BlockSpec((BR, O), lambda i: (i, 0)),
        out_shape=jax.ShapeDtypeStruct((N2P, O), jnp.float32),
    )(acc[0], acc[1], deg[0], deg[1], W2a, b2a, W2b, b2b)


def kernel(x, edge_index0, edge_index1, W1, b1, W2a, b2a, W2b, b2b):
    eidx0, edst0 = _prep_edges(edge_index0, SPLIT0, N1P - 1)
    eidx1, edst1 = _prep_edges(edge_index1, SPLIT1, N2P - 1)

    deg0, deg1 = _make_degrees()(edst0, edst1)
    acc0 = _make_segsum(N1P, SPLIT0)(x, eidx0)
    h = _mlp1(acc0, deg0, W1, b1.reshape(1, D))
    acc1 = _make_segsum(N2P, SPLIT1)(h, eidx1)
    out = _mlp2(acc1, deg1, W2a, b2a.reshape(1, H), W2b, b2b.reshape(1, O))
    return out[:N2]
